# Initial kernel scaffold; baseline (speedup 1.0000x reference)
import jax
import jax.numpy as jnp
from jax.experimental import pallas as pl


def kernel(nfeat, edge_index, efeat, W, b, We, be):
    raise NotImplementedError("write your pallas kernel here")


# speedup vs baseline: 4.6435x; 4.6435x over previous
import functools

import jax
import jax.numpy as jnp
from jax import lax
from jax.experimental import pallas as pl
from jax.experimental.pallas import tpu as pltpu
from jax.experimental.pallas import tpu_sc as plsc

N_NODES = 10000
N_EDGES = 320000
D = 128

NC = 2
NS = 16
CHUNK = 80
EPT = N_EDGES // NS
ITERS = EPT // CHUNK
NPAD = 10240
ROWS_PT = NPAD // NS
CNT_PAD = 10240
CPT = CNT_PAD // NS


def _sc_body(src_h, dst_h, nfeat_h, efeat_h, zeros_h,
             agg_o, esum_o, cnt_o,
             idx_s, idx_d, rows, ones, zb, acc_sh, cnt_sh, sem):
    cid = lax.axis_index("c")
    sid = lax.axis_index("s")

    def _fill(j, _):
        ones[pl.ds(j * 16, 16)] = jnp.ones((16,), jnp.float32)
        return 0
    lax.fori_loop(0, CHUNK // 16, _fill, 0)

    def _zfill(j, _):
        zb[pl.ds(j * 16, 16)] = jnp.zeros((16,), jnp.float32)
        return 0
    lax.fori_loop(0, CPT // 16, _zfill, 0)

    pltpu.sync_copy(zeros_h.at[pl.ds(sid * ROWS_PT, ROWS_PT)],
                    acc_sh.at[pl.ds(sid * ROWS_PT, ROWS_PT)])
    pltpu.sync_copy(zb, cnt_sh.at[pl.ds(sid * CPT, CPT)])
    plsc.subcore_barrier()

    @pl.when(cid == 0)
    def _agg_loop():
        def it(k, carry):
            base = sid * EPT + k * CHUNK
            pltpu.sync_copy(src_h.at[pl.ds(base, CHUNK)], idx_s)
            pltpu.sync_copy(dst_h.at[pl.ds(base, CHUNK)], idx_d)
            pltpu.async_copy(nfeat_h.at[idx_s], rows, sem).wait()
            pltpu.sync_copy(rows, acc_sh.at[idx_d], add=True)
            return carry
        lax.fori_loop(0, ITERS, it, 0)

    @pl.when(cid == 1)
    def _esum_loop():
        def it(k, carry):
            base = sid * EPT + k * CHUNK
            pltpu.sync_copy(dst_h.at[pl.ds(base, CHUNK)], idx_d)
            pltpu.sync_copy(efeat_h.at[pl.ds(base, CHUNK)], rows)
            pltpu.sync_copy(rows, acc_sh.at[idx_d], add=True)
            pltpu.sync_copy(ones, cnt_sh.at[idx_d], add=True)
            return carry
        lax.fori_loop(0, ITERS, it, 0)

    plsc.subcore_barrier()

    @pl.when(cid == 0)
    def _out0():
        pltpu.sync_copy(acc_sh.at[pl.ds(sid * ROWS_PT, ROWS_PT)],
                        agg_o.at[pl.ds(sid * ROWS_PT, ROWS_PT)])

    @pl.when(cid == 1)
    def _out1():
        pltpu.sync_copy(acc_sh.at[pl.ds(sid * ROWS_PT, ROWS_PT)],
                        esum_o.at[pl.ds(sid * ROWS_PT, ROWS_PT)])
        pltpu.sync_copy(cnt_sh.at[pl.ds(sid * CPT, CPT)],
                        cnt_o.at[pl.ds(sid * CPT, CPT)])


_sc_segsum = functools.partial(
    pl.kernel,
    out_type=(
        jax.ShapeDtypeStruct((NPAD, D), jnp.float32),
        jax.ShapeDtypeStruct((NPAD, D), jnp.float32),
        jax.ShapeDtypeStruct((CNT_PAD,), jnp.float32),
    ),
    mesh=plsc.VectorSubcoreMesh(core_axis_name="c", subcore_axis_name="s"),
    scratch_types=[
        pltpu.VMEM((CHUNK,), jnp.int32),
        pltpu.VMEM((CHUNK,), jnp.int32),
        pltpu.VMEM((CHUNK, D), jnp.float32),
        pltpu.VMEM((CHUNK,), jnp.float32),
        pltpu.VMEM((CPT,), jnp.float32),
        pltpu.VMEM_SHARED((NPAD, D), jnp.float32),
        pltpu.VMEM_SHARED((CNT_PAD,), jnp.float32),
        pltpu.SemaphoreType.DMA,
    ],
)(_sc_body)


ROWS_BLK = 1000


def _tc_body(x_ref, a_ref, s_ref, c_ref, w_ref, b_ref, we_ref, be_ref, o_ref):
    c = c_ref[...]
    inv_d = 1.0 / (c + 1.0)
    inv_m = 1.0 / jnp.maximum(c, 1.0)
    x = a_ref[...] + x_ref[...] * inv_d
    y = s_ref[...] * inv_m
    out = jnp.dot(x, w_ref[...], preferred_element_type=jnp.float32)
    out += jnp.dot(y, we_ref[...], preferred_element_type=jnp.float32)
    out += b_ref[...] * (1.0 + inv_d)
    out += be_ref[...] * (c * inv_m)
    o_ref[...] = out


def _tc_combine(nfeat, agg, esum, cnt, W, b, We, be):
    grid = N_NODES // ROWS_BLK
    return pl.pallas_call(
        _tc_body,
        grid=(grid,),
        in_specs=[
            pl.BlockSpec((ROWS_BLK, D), lambda i: (i, 0)),
            pl.BlockSpec((ROWS_BLK, D), lambda i: (i, 0)),
            pl.BlockSpec((ROWS_BLK, D), lambda i: (i, 0)),
            pl.BlockSpec((ROWS_BLK, 1), lambda i: (i, 0)),
            pl.BlockSpec((D, D), lambda i: (0, 0)),
            pl.BlockSpec((1, D), lambda i: (0, 0)),
            pl.BlockSpec((D, D), lambda i: (0, 0)),
            pl.BlockSpec((1, D), lambda i: (0, 0)),
        ],
        out_specs=pl.BlockSpec((ROWS_BLK, D), lambda i: (i, 0)),
        out_shape=jax.ShapeDtypeStruct((N_NODES, D), jnp.float32),
    )(nfeat, agg, esum, cnt, W, b, We, be)


def kernel(nfeat, edge_index, efeat, W, b, We, be):
    src = edge_index[0].astype(jnp.int32)
    dst = edge_index[1].astype(jnp.int32)
    zeros = jnp.zeros((NPAD, D), jnp.float32)
    agg_p, esum_p, cnt_pad = _sc_segsum(src, dst, nfeat, efeat, zeros)
    agg = agg_p[:N_NODES]
    esum = esum_p[:N_NODES]
    cnt = cnt_pad[:N_NODES].reshape(N_NODES, 1)
    return _tc_combine(nfeat, agg, esum, cnt,
                       W, b.reshape(1, D), We, be.reshape(1, D))


# speedup vs baseline: 8.5259x; 1.8361x over previous
import functools

import jax
import jax.numpy as jnp
from jax import lax
from jax.experimental import pallas as pl
from jax.experimental.pallas import tpu as pltpu
from jax.experimental.pallas import tpu_sc as plsc

N_NODES = 10000
N_EDGES = 320000
D = 128

NC = 2
NS = 16
CHUNK = 80
EPT = N_EDGES // NS
ITERS = EPT // CHUNK
BLK_CH = 50
NBLK = ITERS // BLK_CH
HALF = BLK_CH // 2
NPAD = 10240
ROWS_PT = NPAD // NS
CNT_PAD = 10240
CPT = CNT_PAD // NS


def _sc_body(src_h, dst_h, nfeat_h, efeat_h,
             agg_o, esum_o, cnt_o,
             sidx, didx, rows0, rows1, ones, zb, acc_sh, cnt_sh, sem0, sem1):
    cid = lax.axis_index("c")
    sid = lax.axis_index("s")

    def _fill(j, _):
        ones[pl.ds(j * 16, 16)] = jnp.ones((16,), jnp.float32)
        return 0
    lax.fori_loop(0, CHUNK // 16, _fill, 0)

    def _zfill(j, _):
        zb[pl.ds(j * 16, 16)] = jnp.zeros((16,), jnp.float32)
        return 0
    lax.fori_loop(0, CPT // 16, _zfill, 0)

    def _rzfill(j, _):
        def _rz(l, _2):
            rows0[j, pl.ds(l * 16, 16)] = jnp.zeros((16,), jnp.float32)
            return 0
        lax.fori_loop(0, D // 16, _rz, 0)
        return 0
    lax.fori_loop(0, CHUNK, _rzfill, 0)

    def _zacc(j, _):
        pltpu.sync_copy(rows0,
                        acc_sh.at[pl.ds(sid * ROWS_PT + j * CHUNK, CHUNK)])
        return 0
    lax.fori_loop(0, ROWS_PT // CHUNK, _zacc, 0)
    pltpu.sync_copy(zb, cnt_sh.at[pl.ds(sid * CPT, CPT)])
    plsc.subcore_barrier()

    @pl.when(cid == 0)
    def _agg_loop():
        def blk(b, carry):
            pltpu.sync_copy(src_h.at[sid * NBLK + b], sidx)
            pltpu.sync_copy(dst_h.at[sid * NBLK + b], didx)
            pltpu.async_copy(nfeat_h.at[sidx.at[0]], rows0, sem0)

            def it(g, c2):
                k0 = 2 * g
                pltpu.make_async_copy(nfeat_h.at[sidx.at[k0]], rows0, sem0).wait()
                pltpu.async_copy(nfeat_h.at[sidx.at[k0 + 1]], rows1, sem1)
                pltpu.sync_copy(rows0, acc_sh.at[didx.at[k0]], add=True)
                pltpu.make_async_copy(nfeat_h.at[sidx.at[k0 + 1]], rows1, sem1).wait()

                @pl.when(g + 1 < HALF)
                def _():
                    pltpu.async_copy(nfeat_h.at[sidx.at[k0 + 2]], rows0, sem0)

                pltpu.sync_copy(rows1, acc_sh.at[didx.at[k0 + 1]], add=True)
                return c2
            lax.fori_loop(0, HALF, it, 0)
            return carry
        lax.fori_loop(0, NBLK, blk, 0)

    @pl.when(cid == 1)
    def _esum_loop():
        def blk(b, carry):
            pltpu.sync_copy(dst_h.at[sid * NBLK + b], didx)
            base0 = (sid * NBLK + b) * BLK_CH * CHUNK
            pltpu.async_copy(efeat_h.at[pl.ds(base0, CHUNK)], rows0, sem0)

            def it(g, c2):
                k0 = 2 * g
                b0 = base0 + k0 * CHUNK
                pltpu.make_async_copy(efeat_h.at[pl.ds(b0, CHUNK)], rows0, sem0).wait()
                pltpu.async_copy(efeat_h.at[pl.ds(b0 + CHUNK, CHUNK)], rows1, sem1)
                pltpu.sync_copy(rows0, acc_sh.at[didx.at[k0]], add=True)
                pltpu.sync_copy(ones, cnt_sh.at[didx.at[k0]], add=True)
                pltpu.make_async_copy(efeat_h.at[pl.ds(b0 + CHUNK, CHUNK)], rows1, sem1).wait()

                @pl.when(g + 1 < HALF)
                def _():
                    pltpu.async_copy(efeat_h.at[pl.ds(b0 + 2 * CHUNK, CHUNK)], rows0, sem0)

                pltpu.sync_copy(rows1, acc_sh.at[didx.at[k0 + 1]], add=True)
                pltpu.sync_copy(ones, cnt_sh.at[didx.at[k0 + 1]], add=True)
                return c2
            lax.fori_loop(0, HALF, it, 0)
            return carry
        lax.fori_loop(0, NBLK, blk, 0)

    plsc.subcore_barrier()

    @pl.when(cid == 0)
    def _out0():
        pltpu.sync_copy(acc_sh.at[pl.ds(sid * ROWS_PT, ROWS_PT)],
                        agg_o.at[pl.ds(sid * ROWS_PT, ROWS_PT)])

    @pl.when(cid == 1)
    def _out1():
        pltpu.sync_copy(acc_sh.at[pl.ds(sid * ROWS_PT, ROWS_PT)],
                        esum_o.at[pl.ds(sid * ROWS_PT, ROWS_PT)])
        pltpu.sync_copy(cnt_sh.at[pl.ds(sid * CPT, CPT)],
                        cnt_o.at[pl.ds(sid * CPT, CPT)])


_sc_segsum = functools.partial(
    pl.kernel,
    out_type=(
        jax.ShapeDtypeStruct((NPAD, D), jnp.float32),
        jax.ShapeDtypeStruct((NPAD, D), jnp.float32),
        jax.ShapeDtypeStruct((CNT_PAD,), jnp.float32),
    ),
    mesh=plsc.VectorSubcoreMesh(core_axis_name="c", subcore_axis_name="s"),
    scratch_types=[
        pltpu.VMEM((BLK_CH, CHUNK), jnp.int32),
        pltpu.VMEM((BLK_CH, CHUNK), jnp.int32),
        pltpu.VMEM((CHUNK, D), jnp.float32),
        pltpu.VMEM((CHUNK, D), jnp.float32),
        pltpu.VMEM((CHUNK,), jnp.float32),
        pltpu.VMEM((CPT,), jnp.float32),
        pltpu.VMEM_SHARED((NPAD, D), jnp.float32),
        pltpu.VMEM_SHARED((CNT_PAD,), jnp.float32),
        pltpu.SemaphoreType.DMA,
        pltpu.SemaphoreType.DMA,
    ],
)(_sc_body)


ROWS_BLK = 1000


def _tc_body(x_ref, a_ref, s_ref, c_ref, w_ref, b_ref, we_ref, be_ref, o_ref):
    c = c_ref[...]
    inv_d = 1.0 / (c + 1.0)
    inv_m = 1.0 / jnp.maximum(c, 1.0)
    x = a_ref[...] + x_ref[...] * inv_d
    y = s_ref[...] * inv_m
    out = jnp.dot(x, w_ref[...], preferred_element_type=jnp.float32)
    out += jnp.dot(y, we_ref[...], preferred_element_type=jnp.float32)
    out += b_ref[...] * (1.0 + inv_d)
    out += be_ref[...] * (c * inv_m)
    o_ref[...] = out


def _tc_combine(nfeat, agg, esum, cnt, W, b, We, be):
    grid = N_NODES // ROWS_BLK
    return pl.pallas_call(
        _tc_body,
        grid=(grid,),
        in_specs=[
            pl.BlockSpec((ROWS_BLK, D), lambda i: (i, 0)),
            pl.BlockSpec((ROWS_BLK, D), lambda i: (i, 0)),
            pl.BlockSpec((ROWS_BLK, D), lambda i: (i, 0)),
            pl.BlockSpec((ROWS_BLK, 1), lambda i: (i, 0)),
            pl.BlockSpec((D, D), lambda i: (0, 0)),
            pl.BlockSpec((1, D), lambda i: (0, 0)),
            pl.BlockSpec((D, D), lambda i: (0, 0)),
            pl.BlockSpec((1, D), lambda i: (0, 0)),
        ],
        out_specs=pl.BlockSpec((ROWS_BLK, D), lambda i: (i, 0)),
        out_shape=jax.ShapeDtypeStruct((N_NODES, D), jnp.float32),
    )(nfeat, agg, esum, cnt, W, b, We, be)


def kernel(nfeat, edge_index, efeat, W, b, We, be):
    src = edge_index[0].astype(jnp.int32).reshape(NS * NBLK, BLK_CH, CHUNK)
    dst = edge_index[1].astype(jnp.int32).reshape(NS * NBLK, BLK_CH, CHUNK)
    agg_p, esum_p, cnt_pad = _sc_segsum(src, dst, nfeat, efeat)
    agg = agg_p[:N_NODES]
    esum = esum_p[:N_NODES]
    cnt = cnt_pad[:N_NODES].reshape(N_NODES, 1)
    return _tc_combine(nfeat, agg, esum, cnt,
                       W, b.reshape(1, D), We, be.reshape(1, D))


# speedup vs baseline: 8.5402x; 1.0017x over previous
import functools

import jax
import jax.numpy as jnp
from jax import lax
from jax.experimental import pallas as pl
from jax.experimental.pallas import tpu as pltpu
from jax.experimental.pallas import tpu_sc as plsc

N_NODES = 10000
N_EDGES = 320000
D = 128

NC = 2
NS = 16
CHUNK = 128
TAIL = 32
EPT = N_EDGES // NS
MAIN = EPT - TAIL
ITERS = MAIN // CHUNK
BLK_CH = 52
NBLK = ITERS // BLK_CH
HALF = BLK_CH // 2
NPAD = 10240
ROWS_PT = NPAD // NS
CNT_PAD = 10240
CPT = CNT_PAD // NS


def _sc_body(src_h, dst_h, tsrc_h, teid_h, tdst_h, nfeat_h, efeat_h,
             agg_o, esum_o, cnt_o,
             sidx, didx, tsidx, tdidx, rows0, rows1, ones, zb,
             acc_sh, cnt_sh, sem0, sem1):
    cid = lax.axis_index("c")
    sid = lax.axis_index("s")

    def _fill(j, _):
        ones[pl.ds(j * 16, 16)] = jnp.ones((16,), jnp.float32)
        return 0
    lax.fori_loop(0, CHUNK // 16, _fill, 0)

    def _zfill(j, _):
        zb[pl.ds(j * 16, 16)] = jnp.zeros((16,), jnp.float32)
        return 0
    lax.fori_loop(0, CPT // 16, _zfill, 0)

    def _rzfill(j, _):
        def _rz(l, _2):
            rows0[j, pl.ds(l * 16, 16)] = jnp.zeros((16,), jnp.float32)
            return 0
        lax.fori_loop(0, D // 16, _rz, 0)
        return 0
    lax.fori_loop(0, CHUNK, _rzfill, 0)

    def _zacc(j, _):
        pltpu.sync_copy(rows0,
                        acc_sh.at[pl.ds(sid * ROWS_PT + j * CHUNK, CHUNK)])
        return 0
    lax.fori_loop(0, ROWS_PT // CHUNK, _zacc, 0)
    pltpu.sync_copy(zb, cnt_sh.at[pl.ds(sid * CPT, CPT)])
    plsc.subcore_barrier()

    @pl.when(cid == 0)
    def _agg_loop():
        def blk(b, carry):
            pltpu.sync_copy(src_h.at[sid * NBLK + b], sidx)
            pltpu.sync_copy(dst_h.at[sid * NBLK + b], didx)
            pltpu.async_copy(nfeat_h.at[sidx.at[0]], rows0, sem0)

            def it(g, c2):
                k0 = 2 * g
                pltpu.make_async_copy(nfeat_h.at[sidx.at[k0]], rows0, sem0).wait()
                pltpu.async_copy(nfeat_h.at[sidx.at[k0 + 1]], rows1, sem1)
                pltpu.sync_copy(rows0, acc_sh.at[didx.at[k0]], add=True)
                pltpu.make_async_copy(nfeat_h.at[sidx.at[k0 + 1]], rows1, sem1).wait()

                @pl.when(g + 1 < HALF)
                def _():
                    pltpu.async_copy(nfeat_h.at[sidx.at[k0 + 2]], rows0, sem0)

                pltpu.sync_copy(rows1, acc_sh.at[didx.at[k0 + 1]], add=True)
                return c2
            lax.fori_loop(0, HALF, it, 0)
            return carry
        lax.fori_loop(0, NBLK, blk, 0)

        pltpu.sync_copy(tsrc_h.at[sid], tsidx)
        pltpu.sync_copy(tdst_h.at[sid], tdidx)
        pltpu.sync_copy(nfeat_h.at[tsidx.at[0]], rows0)
        pltpu.sync_copy(rows0, acc_sh.at[tdidx.at[0]], add=True)

    @pl.when(cid == 1)
    def _esum_loop():
        def blk(b, carry):
            pltpu.sync_copy(dst_h.at[sid * NBLK + b], didx)
            base0 = sid * EPT + b * BLK_CH * CHUNK
            pltpu.async_copy(efeat_h.at[pl.ds(base0, CHUNK)], rows0, sem0)

            def it(g, c2):
                k0 = 2 * g
                b0 = base0 + k0 * CHUNK
                pltpu.make_async_copy(efeat_h.at[pl.ds(b0, CHUNK)], rows0, sem0).wait()
                pltpu.async_copy(efeat_h.at[pl.ds(b0 + CHUNK, CHUNK)], rows1, sem1)
                pltpu.sync_copy(rows0, acc_sh.at[didx.at[k0]], add=True)
                pltpu.sync_copy(ones, cnt_sh.at[didx.at[k0]], add=True)
                pltpu.make_async_copy(efeat_h.at[pl.ds(b0 + CHUNK, CHUNK)], rows1, sem1).wait()

                @pl.when(g + 1 < HALF)
                def _():
                    pltpu.async_copy(efeat_h.at[pl.ds(b0 + 2 * CHUNK, CHUNK)], rows0, sem0)

                pltpu.sync_copy(rows1, acc_sh.at[didx.at[k0 + 1]], add=True)
                pltpu.sync_copy(ones, cnt_sh.at[didx.at[k0 + 1]], add=True)
                return c2
            lax.fori_loop(0, HALF, it, 0)
            return carry
        lax.fori_loop(0, NBLK, blk, 0)

        pltpu.sync_copy(teid_h.at[sid], tsidx)
        pltpu.sync_copy(tdst_h.at[sid], tdidx)
        pltpu.sync_copy(efeat_h.at[tsidx.at[0]], rows0)
        pltpu.sync_copy(rows0, acc_sh.at[tdidx.at[0]], add=True)
        pltpu.sync_copy(ones, cnt_sh.at[tdidx.at[0]], add=True)

    plsc.subcore_barrier()

    @pl.when(cid == 0)
    def _out0():
        pltpu.sync_copy(acc_sh.at[pl.ds(sid * ROWS_PT, ROWS_PT)],
                        agg_o.at[pl.ds(sid * ROWS_PT, ROWS_PT)])

    @pl.when(cid == 1)
    def _out1():
        pltpu.sync_copy(acc_sh.at[pl.ds(sid * ROWS_PT, ROWS_PT)],
                        esum_o.at[pl.ds(sid * ROWS_PT, ROWS_PT)])
        pltpu.sync_copy(cnt_sh.at[pl.ds(sid * CPT, CPT)],
                        cnt_o.at[pl.ds(sid * CPT, CPT)])


_sc_segsum = functools.partial(
    pl.kernel,
    out_type=(
        jax.ShapeDtypeStruct((NPAD, D), jnp.float32),
        jax.ShapeDtypeStruct((NPAD, D), jnp.float32),
        jax.ShapeDtypeStruct((CNT_PAD,), jnp.float32),
    ),
    mesh=plsc.VectorSubcoreMesh(core_axis_name="c", subcore_axis_name="s"),
    scratch_types=[
        pltpu.VMEM((BLK_CH, CHUNK), jnp.int32),
        pltpu.VMEM((BLK_CH, CHUNK), jnp.int32),
        pltpu.VMEM((1, CHUNK), jnp.int32),
        pltpu.VMEM((1, CHUNK), jnp.int32),
        pltpu.VMEM((CHUNK, D), jnp.float32),
        pltpu.VMEM((CHUNK, D), jnp.float32),
        pltpu.VMEM((CHUNK,), jnp.float32),
        pltpu.VMEM((CPT,), jnp.float32),
        pltpu.VMEM_SHARED((NPAD, D), jnp.float32),
        pltpu.VMEM_SHARED((CNT_PAD,), jnp.float32),
        pltpu.SemaphoreType.DMA,
        pltpu.SemaphoreType.DMA,
    ],
)(_sc_body)


ROWS_BLK = 1000


def _tc_body(x_ref, a_ref, s_ref, c_ref, w_ref, b_ref, we_ref, be_ref, o_ref):
    c = c_ref[...]
    inv_d = 1.0 / (c + 1.0)
    inv_m = 1.0 / jnp.maximum(c, 1.0)
    x = a_ref[...] + x_ref[...] * inv_d
    y = s_ref[...] * inv_m
    out = jnp.dot(x, w_ref[...], preferred_element_type=jnp.float32)
    out += jnp.dot(y, we_ref[...], preferred_element_type=jnp.float32)
    out += b_ref[...] * (1.0 + inv_d)
    out += be_ref[...] * (c * inv_m)
    o_ref[...] = out


def _tc_combine(nfeat, agg, esum, cnt, W, b, We, be):
    grid = N_NODES // ROWS_BLK
    return pl.pallas_call(
        _tc_body,
        grid=(grid,),
        in_specs=[
            pl.BlockSpec((ROWS_BLK, D), lambda i: (i, 0)),
            pl.BlockSpec((ROWS_BLK, D), lambda i: (i, 0)),
            pl.BlockSpec((ROWS_BLK, D), lambda i: (i, 0)),
            pl.BlockSpec((ROWS_BLK, 1), lambda i: (i, 0)),
            pl.BlockSpec((D, D), lambda i: (0, 0)),
            pl.BlockSpec((1, D), lambda i: (0, 0)),
            pl.BlockSpec((D, D), lambda i: (0, 0)),
            pl.BlockSpec((1, D), lambda i: (0, 0)),
        ],
        out_specs=pl.BlockSpec((ROWS_BLK, D), lambda i: (i, 0)),
        out_shape=jax.ShapeDtypeStruct((N_NODES, D), jnp.float32),
    )(nfeat, agg, esum, cnt, W, b, We, be)


def kernel(nfeat, edge_index, efeat, W, b, We, be):
    src = edge_index[0].astype(jnp.int32).reshape(NS, EPT)
    dst = edge_index[1].astype(jnp.int32).reshape(NS, EPT)
    src_m = src[:, :MAIN].reshape(NS * NBLK, BLK_CH, CHUNK)
    dst_m = dst[:, :MAIN].reshape(NS * NBLK, BLK_CH, CHUNK)
    pad_i = jnp.zeros((NS, 1, CHUNK - TAIL), jnp.int32)
    pad_d = jnp.full((NS, 1, CHUNK - TAIL), N_NODES, jnp.int32)
    src_t = jnp.concatenate([src[:, MAIN:].reshape(NS, 1, TAIL), pad_i], -1)
    dst_t = jnp.concatenate([dst[:, MAIN:].reshape(NS, 1, TAIL), pad_d], -1)
    eid_t = jnp.concatenate(
        [(jnp.arange(NS, dtype=jnp.int32) * EPT + MAIN)[:, None, None]
         + jnp.arange(TAIL, dtype=jnp.int32)[None, None, :], pad_i], -1)
    agg_p, esum_p, cnt_pad = _sc_segsum(src_m, dst_m, src_t, eid_t, dst_t,
                                        nfeat, efeat)
    agg = agg_p[:N_NODES]
    esum = esum_p[:N_NODES]
    cnt = cnt_pad[:N_NODES].reshape(N_NODES, 1)
    return _tc_combine(nfeat, agg, esum, cnt,
                       W, b.reshape(1, D), We, be.reshape(1, D))


# speedup vs baseline: 9.8371x; 1.1519x over previous
import functools

import jax
import jax.numpy as jnp
from jax import lax
from jax.experimental import pallas as pl
from jax.experimental.pallas import tpu as pltpu
from jax.experimental.pallas import tpu_sc as plsc

N_NODES = 10000
N_EDGES = 320000
D = 128

NC = 2
NS = 16
CHUNK = 64
TAIL = 32
EPT = N_EDGES // NS
MAIN = EPT - TAIL
ITERS = MAIN // CHUNK
NBUF = 4
BLK_CH = 52
NBLK = ITERS // BLK_CH
OUTER = BLK_CH // NBUF
NPAD = 10240
ROWS_PT = NPAD // NS
CNT_PAD = 10240
CPT = CNT_PAD // NS


def _sc_body(src_h, dst_h, tsrc_h, teid_h, tdst_h, nfeat_h, efeat_h,
             agg_o, esum_o, cnt_o,
             sidx, didx, tsidx, tdidx, r0, r1, r2, r3, ones, zb,
             acc_sh, cnt_sh,
             g0, g1, g2, g3, s0, s1, s2, s3, csem):
    cid = lax.axis_index("c")
    sid = lax.axis_index("s")
    rows = (r0, r1, r2, r3)
    gsem = (g0, g1, g2, g3)
    ssem = (s0, s1, s2, s3)

    def _fill(j, _):
        ones[pl.ds(j * 16, 16)] = jnp.ones((16,), jnp.float32)
        return 0
    lax.fori_loop(0, CHUNK // 16, _fill, 0)

    def _zfill(j, _):
        zb[pl.ds(j * 16, 16)] = jnp.zeros((16,), jnp.float32)
        return 0
    lax.fori_loop(0, CPT // 16, _zfill, 0)

    def _rzfill(j, _):
        def _rz(l, _2):
            r0[j, pl.ds(l * 16, 16)] = jnp.zeros((16,), jnp.float32)
            return 0
        lax.fori_loop(0, D // 16, _rz, 0)
        return 0
    lax.fori_loop(0, CHUNK, _rzfill, 0)

    def _zacc(j, _):
        pltpu.sync_copy(r0,
                        acc_sh.at[pl.ds(sid * ROWS_PT + j * CHUNK, CHUNK)])
        return 0
    lax.fori_loop(0, ROWS_PT // CHUNK, _zacc, 0)
    pltpu.sync_copy(zb, cnt_sh.at[pl.ds(sid * CPT, CPT)])
    plsc.subcore_barrier()

    @pl.when(cid == 0)
    def _agg_loop():
        def blk(b, carry):
            pltpu.sync_copy(src_h.at[sid * NBLK + b], sidx)
            pltpu.sync_copy(dst_h.at[sid * NBLK + b], didx)
            pltpu.async_copy(nfeat_h.at[sidx.at[0]], r0, g0)
            pltpu.async_copy(nfeat_h.at[sidx.at[1]], r1, g1)

            def outer(i, c2):
                for j in range(NBUF):
                    k = i * NBUF + j
                    j2 = (j + 2) % NBUF
                    pltpu.make_async_copy(nfeat_h.at[sidx.at[k]],
                                          rows[j], gsem[j]).wait()
                    pltpu.async_copy(rows[j], acc_sh.at[didx.at[k]],
                                     ssem[j], add=True)

                    @pl.when(k + 2 < BLK_CH)
                    def _refill():
                        @pl.when(k >= 2)
                        def _():
                            pltpu.make_async_copy(rows[j2],
                                                  acc_sh.at[didx.at[0]],
                                                  ssem[j2]).wait()
                        pltpu.async_copy(nfeat_h.at[sidx.at[k + 2]],
                                         rows[j2], gsem[j2])
                return c2
            lax.fori_loop(0, OUTER, outer, 0)

            for j in range(NBUF):
                pltpu.make_async_copy(rows[j], acc_sh.at[didx.at[0]],
                                      ssem[j]).wait()
            return carry
        lax.fori_loop(0, NBLK, blk, 0)

        pltpu.sync_copy(tsrc_h.at[sid], tsidx)
        pltpu.sync_copy(tdst_h.at[sid], tdidx)
        pltpu.sync_copy(nfeat_h.at[tsidx.at[0]], r0)
        pltpu.sync_copy(r0, acc_sh.at[tdidx.at[0]], add=True)

    @pl.when(cid == 1)
    def _esum_loop():
        def blk(b, carry):
            pltpu.sync_copy(dst_h.at[sid * NBLK + b], didx)
            base = sid * EPT + b * BLK_CH * CHUNK
            pltpu.async_copy(efeat_h.at[pl.ds(base, CHUNK)], r0, g0)
            pltpu.async_copy(efeat_h.at[pl.ds(base + CHUNK, CHUNK)], r1, g1)

            def outer(i, c2):
                for j in range(NBUF):
                    k = i * NBUF + j
                    j2 = (j + 2) % NBUF
                    pltpu.make_async_copy(
                        efeat_h.at[pl.ds(base + k * CHUNK, CHUNK)],
                        rows[j], gsem[j]).wait()
                    pltpu.async_copy(rows[j], acc_sh.at[didx.at[k]],
                                     ssem[j], add=True)
                    pltpu.async_copy(ones, cnt_sh.at[didx.at[k]], csem,
                                     add=True)

                    @pl.when(k + 2 < BLK_CH)
                    def _refill():
                        @pl.when(k >= 2)
                        def _():
                            pltpu.make_async_copy(rows[j2],
                                                  acc_sh.at[didx.at[0]],
                                                  ssem[j2]).wait()
                        pltpu.async_copy(
                            efeat_h.at[pl.ds(base + (k + 2) * CHUNK, CHUNK)],
                            rows[j2], gsem[j2])
                return c2
            lax.fori_loop(0, OUTER, outer, 0)

            for j in range(NBUF):
                pltpu.make_async_copy(rows[j], acc_sh.at[didx.at[0]],
                                      ssem[j]).wait()

            def _cdrain(k, c2):
                pltpu.make_async_copy(ones, cnt_sh.at[didx.at[0]],
                                      csem).wait()
                return c2
            lax.fori_loop(0, BLK_CH, _cdrain, 0)
            return carry
        lax.fori_loop(0, NBLK, blk, 0)

        pltpu.sync_copy(teid_h.at[sid], tsidx)
        pltpu.sync_copy(tdst_h.at[sid], tdidx)
        pltpu.sync_copy(efeat_h.at[tsidx.at[0]], r0)
        pltpu.sync_copy(r0, acc_sh.at[tdidx.at[0]], add=True)
        pltpu.sync_copy(ones, cnt_sh.at[tdidx.at[0]], add=True)

    plsc.subcore_barrier()

    @pl.when(cid == 0)
    def _out0():
        pltpu.sync_copy(acc_sh.at[pl.ds(sid * ROWS_PT, ROWS_PT)],
                        agg_o.at[pl.ds(sid * ROWS_PT, ROWS_PT)])

    @pl.when(cid == 1)
    def _out1():
        pltpu.sync_copy(acc_sh.at[pl.ds(sid * ROWS_PT, ROWS_PT)],
                        esum_o.at[pl.ds(sid * ROWS_PT, ROWS_PT)])
        pltpu.sync_copy(cnt_sh.at[pl.ds(sid * CPT, CPT)],
                        cnt_o.at[pl.ds(sid * CPT, CPT)])


_sc_segsum = functools.partial(
    pl.kernel,
    out_type=(
        jax.ShapeDtypeStruct((NPAD, D), jnp.float32),
        jax.ShapeDtypeStruct((NPAD, D), jnp.float32),
        jax.ShapeDtypeStruct((CNT_PAD,), jnp.float32),
    ),
    mesh=plsc.VectorSubcoreMesh(core_axis_name="c", subcore_axis_name="s"),
    scratch_types=[
        pltpu.VMEM((BLK_CH, CHUNK), jnp.int32),
        pltpu.VMEM((BLK_CH, CHUNK), jnp.int32),
        pltpu.VMEM((1, CHUNK), jnp.int32),
        pltpu.VMEM((1, CHUNK), jnp.int32),
        pltpu.VMEM((CHUNK, D), jnp.float32),
        pltpu.VMEM((CHUNK, D), jnp.float32),
        pltpu.VMEM((CHUNK, D), jnp.float32),
        pltpu.VMEM((CHUNK, D), jnp.float32),
        pltpu.VMEM((CHUNK,), jnp.float32),
        pltpu.VMEM((CPT,), jnp.float32),
        pltpu.VMEM_SHARED((NPAD, D), jnp.float32),
        pltpu.VMEM_SHARED((CNT_PAD,), jnp.float32),
        pltpu.SemaphoreType.DMA,
        pltpu.SemaphoreType.DMA,
        pltpu.SemaphoreType.DMA,
        pltpu.SemaphoreType.DMA,
        pltpu.SemaphoreType.DMA,
        pltpu.SemaphoreType.DMA,
        pltpu.SemaphoreType.DMA,
        pltpu.SemaphoreType.DMA,
        pltpu.SemaphoreType.DMA,
    ],
)(_sc_body)


ROWS_BLK = 1000


def _tc_body(x_ref, a_ref, s_ref, c_ref, w_ref, b_ref, we_ref, be_ref, o_ref):
    c = c_ref[...]
    inv_d = 1.0 / (c + 1.0)
    inv_m = 1.0 / jnp.maximum(c, 1.0)
    x = a_ref[...] + x_ref[...] * inv_d
    y = s_ref[...] * inv_m
    out = jnp.dot(x, w_ref[...], preferred_element_type=jnp.float32)
    out += jnp.dot(y, we_ref[...], preferred_element_type=jnp.float32)
    out += b_ref[...] * (1.0 + inv_d)
    out += be_ref[...] * (c * inv_m)
    o_ref[...] = out


def _tc_combine(nfeat, agg, esum, cnt, W, b, We, be):
    grid = N_NODES // ROWS_BLK
    return pl.pallas_call(
        _tc_body,
        grid=(grid,),
        in_specs=[
            pl.BlockSpec((ROWS_BLK, D), lambda i: (i, 0)),
            pl.BlockSpec((ROWS_BLK, D), lambda i: (i, 0)),
            pl.BlockSpec((ROWS_BLK, D), lambda i: (i, 0)),
            pl.BlockSpec((ROWS_BLK, 1), lambda i: (i, 0)),
            pl.BlockSpec((D, D), lambda i: (0, 0)),
            pl.BlockSpec((1, D), lambda i: (0, 0)),
            pl.BlockSpec((D, D), lambda i: (0, 0)),
            pl.BlockSpec((1, D), lambda i: (0, 0)),
        ],
        out_specs=pl.BlockSpec((ROWS_BLK, D), lambda i: (i, 0)),
        out_shape=jax.ShapeDtypeStruct((N_NODES, D), jnp.float32),
    )(nfeat, agg, esum, cnt, W, b, We, be)


def kernel(nfeat, edge_index, efeat, W, b, We, be):
    src = edge_index[0].astype(jnp.int32).reshape(NS, EPT)
    dst = edge_index[1].astype(jnp.int32).reshape(NS, EPT)
    src_m = src[:, :MAIN].reshape(NS * NBLK, BLK_CH, CHUNK)
    dst_m = dst[:, :MAIN].reshape(NS * NBLK, BLK_CH, CHUNK)
    pad_i = jnp.zeros((NS, 1, CHUNK - TAIL), jnp.int32)
    pad_d = jnp.full((NS, 1, CHUNK - TAIL), N_NODES, jnp.int32)
    src_t = jnp.concatenate([src[:, MAIN:].reshape(NS, 1, TAIL), pad_i], -1)
    dst_t = jnp.concatenate([dst[:, MAIN:].reshape(NS, 1, TAIL), pad_d], -1)
    eid_t = jnp.concatenate(
        [(jnp.arange(NS, dtype=jnp.int32) * EPT + MAIN)[:, None, None]
         + jnp.arange(TAIL, dtype=jnp.int32)[None, None, :], pad_i], -1)
    agg_p, esum_p, cnt_pad = _sc_segsum(src_m, dst_m, src_t, eid_t, dst_t,
                                        nfeat, efeat)
    agg = agg_p[:N_NODES]
    esum = esum_p[:N_NODES]
    cnt = cnt_pad[:N_NODES].reshape(N_NODES, 1)
    return _tc_combine(nfeat, agg, esum, cnt,
                       W, b.reshape(1, D), We, be.reshape(1, D))


# speedup vs baseline: 10.0768x; 1.0244x over previous
import functools

import jax
import jax.numpy as jnp
from jax import lax
from jax.experimental import pallas as pl
from jax.experimental.pallas import tpu as pltpu
from jax.experimental.pallas import tpu_sc as plsc

N_NODES = 10000
N_EDGES = 320000
D = 128

NC = 2
NS = 16
CHUNK = 64
TAIL = 32
EPT = N_EDGES // NS
MAIN = EPT - TAIL
ITERS = MAIN // CHUNK
NBUF = 4
BLK_CH = 52
NBLK = ITERS // BLK_CH
OUTER = BLK_CH // NBUF
NPAD = 10240
ROWS_PT = NPAD // NS
CNT_PAD = 10240
CPT = CNT_PAD // NS


def _sc_body(src_h, dst_h, tsrc_h, teid_h, tdst_h, nfeat_h, efeat_h,
             agg_o, esum_o, cnt_o,
             sidx, didx, tsidx, tdidx, r0, r1, r2, r3, ones, zb,
             acc_sh, cnt_sh,
             g0, g1, g2, g3, s0, s1, s2, s3, csem):
    cid = lax.axis_index("c")
    sid = lax.axis_index("s")
    rows = (r0, r1, r2, r3)
    gsem = (g0, g1, g2, g3)
    ssem = (s0, s1, s2, s3)

    def _fill(j, _):
        ones[pl.ds(j * 16, 16)] = jnp.ones((16,), jnp.float32)
        return 0
    lax.fori_loop(0, CHUNK // 16, _fill, 0)

    def _zfill(j, _):
        zb[pl.ds(j * 16, 16)] = jnp.zeros((16,), jnp.float32)
        return 0
    lax.fori_loop(0, CPT // 16, _zfill, 0)

    def _rzfill(j, _):
        def _rz(l, _2):
            r0[j, pl.ds(l * 16, 16)] = jnp.zeros((16,), jnp.float32)
            return 0
        lax.fori_loop(0, D // 16, _rz, 0)
        return 0
    lax.fori_loop(0, CHUNK, _rzfill, 0)

    def _zacc(j, _):
        pltpu.sync_copy(r0,
                        acc_sh.at[pl.ds(sid * ROWS_PT + j * CHUNK, CHUNK)])
        return 0
    lax.fori_loop(0, ROWS_PT // CHUNK, _zacc, 0)
    pltpu.sync_copy(zb, cnt_sh.at[pl.ds(sid * CPT, CPT)])
    plsc.subcore_barrier()

    @pl.when(cid == 0)
    def _agg_loop():
        def blk(b, carry):
            pltpu.sync_copy(src_h.at[sid * NBLK + b], sidx)
            pltpu.sync_copy(dst_h.at[sid * NBLK + b], didx)
            pltpu.async_copy(nfeat_h.at[sidx.at[0]], r0, g0)
            pltpu.async_copy(nfeat_h.at[sidx.at[1]], r1, g1)

            def outer(i, c2):
                for j in range(NBUF):
                    k = i * NBUF + j
                    j2 = (j + 2) % NBUF
                    pltpu.make_async_copy(nfeat_h.at[sidx.at[k]],
                                          rows[j], gsem[j]).wait()
                    pltpu.async_copy(rows[j], acc_sh.at[didx.at[k]],
                                     ssem[j], add=True)

                    @pl.when(k + 2 < BLK_CH)
                    def _refill():
                        @pl.when(k >= 2)
                        def _():
                            pltpu.make_async_copy(rows[j2],
                                                  acc_sh.at[didx.at[0]],
                                                  ssem[j2]).wait()
                        pltpu.async_copy(nfeat_h.at[sidx.at[k + 2]],
                                         rows[j2], gsem[j2])
                return c2
            lax.fori_loop(0, OUTER, outer, 0)

            for j in range(NBUF):
                pltpu.make_async_copy(rows[j], acc_sh.at[didx.at[0]],
                                      ssem[j]).wait()
            return carry
        lax.fori_loop(0, NBLK, blk, 0)

        pltpu.sync_copy(tsrc_h.at[sid], tsidx)
        pltpu.sync_copy(tdst_h.at[sid], tdidx)
        pltpu.sync_copy(nfeat_h.at[tsidx.at[0]], r0)
        pltpu.sync_copy(r0, acc_sh.at[tdidx.at[0]], add=True)

    @pl.when(cid == 1)
    def _esum_loop():
        def blk(b, carry):
            pltpu.sync_copy(dst_h.at[sid * NBLK + b], didx)
            base = sid * EPT + b * BLK_CH * CHUNK
            pltpu.async_copy(efeat_h.at[pl.ds(base, CHUNK)], r0, g0)
            pltpu.async_copy(efeat_h.at[pl.ds(base + CHUNK, CHUNK)], r1, g1)

            def outer(i, c2):
                for j in range(NBUF):
                    k = i * NBUF + j
                    j2 = (j + 2) % NBUF
                    pltpu.make_async_copy(
                        efeat_h.at[pl.ds(base + k * CHUNK, CHUNK)],
                        rows[j], gsem[j]).wait()
                    pltpu.async_copy(rows[j], acc_sh.at[didx.at[k]],
                                     ssem[j], add=True)
                    pltpu.async_copy(ones, cnt_sh.at[didx.at[k]], csem,
                                     add=True)

                    @pl.when(k + 2 < BLK_CH)
                    def _refill():
                        @pl.when(k >= 2)
                        def _():
                            pltpu.make_async_copy(rows[j2],
                                                  acc_sh.at[didx.at[0]],
                                                  ssem[j2]).wait()
                        pltpu.async_copy(
                            efeat_h.at[pl.ds(base + (k + 2) * CHUNK, CHUNK)],
                            rows[j2], gsem[j2])
                return c2
            lax.fori_loop(0, OUTER, outer, 0)

            for j in range(NBUF):
                pltpu.make_async_copy(rows[j], acc_sh.at[didx.at[0]],
                                      ssem[j]).wait()

            def _cdrain(k, c2):
                pltpu.make_async_copy(ones, cnt_sh.at[didx.at[0]],
                                      csem).wait()
                return c2
            lax.fori_loop(0, BLK_CH, _cdrain, 0)
            return carry
        lax.fori_loop(0, NBLK, blk, 0)

        pltpu.sync_copy(teid_h.at[sid], tsidx)
        pltpu.sync_copy(tdst_h.at[sid], tdidx)
        pltpu.sync_copy(efeat_h.at[tsidx.at[0]], r0)
        pltpu.sync_copy(r0, acc_sh.at[tdidx.at[0]], add=True)
        pltpu.sync_copy(ones, cnt_sh.at[tdidx.at[0]], add=True)

    plsc.subcore_barrier()

    @pl.when(cid == 0)
    def _out0():
        pltpu.sync_copy(acc_sh.at[pl.ds(sid * ROWS_PT, ROWS_PT)],
                        agg_o.at[pl.ds(sid * ROWS_PT, ROWS_PT)])

    @pl.when(cid == 1)
    def _out1():
        pltpu.sync_copy(acc_sh.at[pl.ds(sid * ROWS_PT, ROWS_PT)],
                        esum_o.at[pl.ds(sid * ROWS_PT, ROWS_PT)])
        pltpu.sync_copy(cnt_sh.at[pl.ds(sid * CPT, CPT)],
                        cnt_o.at[pl.ds(sid * CPT, CPT)])


_sc_segsum = functools.partial(
    pl.kernel,
    out_type=(
        jax.ShapeDtypeStruct((NPAD, D), jnp.float32),
        jax.ShapeDtypeStruct((NPAD, D), jnp.float32),
        jax.ShapeDtypeStruct((CNT_PAD,), jnp.float32),
    ),
    mesh=plsc.VectorSubcoreMesh(core_axis_name="c", subcore_axis_name="s"),
    scratch_types=[
        pltpu.VMEM((BLK_CH, CHUNK), jnp.int32),
        pltpu.VMEM((BLK_CH, CHUNK), jnp.int32),
        pltpu.VMEM((1, CHUNK), jnp.int32),
        pltpu.VMEM((1, CHUNK), jnp.int32),
        pltpu.VMEM((CHUNK, D), jnp.float32),
        pltpu.VMEM((CHUNK, D), jnp.float32),
        pltpu.VMEM((CHUNK, D), jnp.float32),
        pltpu.VMEM((CHUNK, D), jnp.float32),
        pltpu.VMEM((CHUNK,), jnp.float32),
        pltpu.VMEM((CPT,), jnp.float32),
        pltpu.VMEM_SHARED((NPAD, D), jnp.float32),
        pltpu.VMEM_SHARED((CNT_PAD,), jnp.float32),
        pltpu.SemaphoreType.DMA,
        pltpu.SemaphoreType.DMA,
        pltpu.SemaphoreType.DMA,
        pltpu.SemaphoreType.DMA,
        pltpu.SemaphoreType.DMA,
        pltpu.SemaphoreType.DMA,
        pltpu.SemaphoreType.DMA,
        pltpu.SemaphoreType.DMA,
        pltpu.SemaphoreType.DMA,
    ],
)(_sc_body)


ROWS_BLK = 1000


def _tc_body(x_ref, a_ref, s_ref, c_ref, w_ref, b_ref, we_ref, be_ref, o_ref):
    c = c_ref[...]
    inv_d = 1.0 / (c + 1.0)
    inv_m = 1.0 / jnp.maximum(c, 1.0)
    x = a_ref[...] + x_ref[...] * inv_d
    y = s_ref[...] * inv_m
    out = jnp.dot(x, w_ref[...], preferred_element_type=jnp.float32)
    out += jnp.dot(y, we_ref[...], preferred_element_type=jnp.float32)
    out += b_ref[...] * (1.0 + inv_d)
    out += be_ref[...] * (c * inv_m)
    o_ref[...] = out


def _tc_combine(nfeat, agg_p, esum_p, cnt, W, b, We, be):
    grid = N_NODES // ROWS_BLK
    return pl.pallas_call(
        _tc_body,
        grid=(grid,),
        in_specs=[
            pl.BlockSpec((ROWS_BLK, D), lambda i: (i, 0)),
            pl.BlockSpec((ROWS_BLK, D), lambda i: (i, 0)),
            pl.BlockSpec((ROWS_BLK, D), lambda i: (i, 0)),
            pl.BlockSpec((ROWS_BLK, 1), lambda i: (i, 0)),
            pl.BlockSpec((D, D), lambda i: (0, 0)),
            pl.BlockSpec((1, D), lambda i: (0, 0)),
            pl.BlockSpec((D, D), lambda i: (0, 0)),
            pl.BlockSpec((1, D), lambda i: (0, 0)),
        ],
        out_specs=pl.BlockSpec((ROWS_BLK, D), lambda i: (i, 0)),
        out_shape=jax.ShapeDtypeStruct((N_NODES, D), jnp.float32),
    )(nfeat, agg_p, esum_p, cnt, W, b, We, be)


def kernel(nfeat, edge_index, efeat, W, b, We, be):
    src = edge_index[0].astype(jnp.int32).reshape(NS, EPT)
    dst = edge_index[1].astype(jnp.int32).reshape(NS, EPT)
    src_m = src[:, :MAIN].reshape(NS * NBLK, BLK_CH, CHUNK)
    dst_m = dst[:, :MAIN].reshape(NS * NBLK, BLK_CH, CHUNK)
    pad_i = jnp.zeros((NS, 1, CHUNK - TAIL), jnp.int32)
    pad_d = jnp.full((NS, 1, CHUNK - TAIL), N_NODES, jnp.int32)
    src_t = jnp.concatenate([src[:, MAIN:].reshape(NS, 1, TAIL), pad_i], -1)
    dst_t = jnp.concatenate([dst[:, MAIN:].reshape(NS, 1, TAIL), pad_d], -1)
    eid_t = jnp.concatenate(
        [(jnp.arange(NS, dtype=jnp.int32) * EPT + MAIN)[:, None, None]
         + jnp.arange(TAIL, dtype=jnp.int32)[None, None, :], pad_i], -1)
    agg_p, esum_p, cnt_pad = _sc_segsum(src_m, dst_m, src_t, eid_t, dst_t,
                                        nfeat, efeat)
    cnt = cnt_pad[:N_NODES].reshape(N_NODES, 1)
    return _tc_combine(nfeat, agg_p, esum_p, cnt,
                       W, b.reshape(1, D), We, be.reshape(1, D))


# speedup vs baseline: 10.5284x; 1.0448x over previous
import functools

import jax
import jax.numpy as jnp
from jax import lax
from jax.experimental import pallas as pl
from jax.experimental.pallas import tpu as pltpu
from jax.experimental.pallas import tpu_sc as plsc

N_NODES = 10000
N_EDGES = 320000
D = 128

NC = 2
NS = 16
CHUNK = 32
TAIL = 32
EPT = N_EDGES // NS
MAIN = EPT - TAIL
ITERS = MAIN // CHUNK
NBUF = 8
LEAD = NBUF // 2
BLK_CH = 48
NBLK = ITERS // BLK_CH
OUTER = BLK_CH // NBUF
NPAD = 10240
ROWS_PT = NPAD // NS
CNT_PAD = 10240
CPT = CNT_PAD // NS


def _sc_body(src_h, dst_h, tsrc_h, tdst_h, nfeat_h, efeat_h,
             agg_o, esum_o, cnt_o,
             sidx, didx, tsidx, tdidx,
             r0, r1, r2, r3, r4, r5, r6, r7, ones, zb,
             acc_sh, cnt_sh,
             g0, g1, g2, g3, g4, g5, g6, g7,
             s0, s1, s2, s3, s4, s5, s6, s7, csem):
    cid = lax.axis_index("c")
    sid = lax.axis_index("s")
    rows = (r0, r1, r2, r3, r4, r5, r6, r7)
    gsem = (g0, g1, g2, g3, g4, g5, g6, g7)
    ssem = (s0, s1, s2, s3, s4, s5, s6, s7)

    def _fill(j, _):
        ones[pl.ds(j * 16, 16)] = jnp.ones((16,), jnp.float32)
        return 0
    lax.fori_loop(0, CHUNK // 16, _fill, 0)

    def _zfill(j, _):
        zb[pl.ds(j * 16, 16)] = jnp.zeros((16,), jnp.float32)
        return 0
    lax.fori_loop(0, CPT // 16, _zfill, 0)

    def _rzfill(j, _):
        def _rz(l, _2):
            r0[j, pl.ds(l * 16, 16)] = jnp.zeros((16,), jnp.float32)
            return 0
        lax.fori_loop(0, D // 16, _rz, 0)
        return 0
    lax.fori_loop(0, CHUNK, _rzfill, 0)

    def _zacc(j, _):
        pltpu.sync_copy(r0,
                        acc_sh.at[pl.ds(sid * ROWS_PT + j * CHUNK, CHUNK)])
        return 0
    lax.fori_loop(0, ROWS_PT // CHUNK, _zacc, 0)
    pltpu.sync_copy(zb, cnt_sh.at[pl.ds(sid * CPT, CPT)])
    plsc.subcore_barrier()

    @pl.when(cid == 0)
    def _agg_loop():
        def blk(b, carry):
            pltpu.sync_copy(src_h.at[sid * NBLK + b], sidx)
            pltpu.sync_copy(dst_h.at[sid * NBLK + b], didx)
            for j in range(LEAD):
                pltpu.async_copy(nfeat_h.at[sidx.at[j]], rows[j], gsem[j])

            def outer(i, c2):
                for j in range(NBUF):
                    k = i * NBUF + j
                    j2 = (j + LEAD) % NBUF
                    pltpu.make_async_copy(nfeat_h.at[sidx.at[k]],
                                          rows[j], gsem[j]).wait()
                    pltpu.async_copy(rows[j], acc_sh.at[didx.at[k]],
                                     ssem[j], add=True)

                    @pl.when(k + LEAD < BLK_CH)
                    def _refill():
                        @pl.when(k >= LEAD)
                        def _():
                            pltpu.make_async_copy(rows[j2],
                                                  acc_sh.at[didx.at[0]],
                                                  ssem[j2]).wait()
                        pltpu.async_copy(nfeat_h.at[sidx.at[k + LEAD]],
                                         rows[j2], gsem[j2])
                return c2
            lax.fori_loop(0, OUTER, outer, 0)

            for j in range(NBUF):
                pltpu.make_async_copy(rows[j], acc_sh.at[didx.at[0]],
                                      ssem[j]).wait()
            return carry
        lax.fori_loop(0, NBLK, blk, 0)

        pltpu.sync_copy(tsrc_h.at[sid], tsidx)
        pltpu.sync_copy(tdst_h.at[sid], tdidx)
        pltpu.sync_copy(nfeat_h.at[tsidx.at[0]], r0)
        pltpu.sync_copy(r0, acc_sh.at[tdidx.at[0]], add=True)

    @pl.when(cid == 1)
    def _esum_loop():
        def blk(b, carry):
            pltpu.sync_copy(dst_h.at[sid * NBLK + b], didx)
            base = sid * EPT + b * BLK_CH * CHUNK
            for j in range(LEAD):
                pltpu.async_copy(efeat_h.at[pl.ds(base + j * CHUNK, CHUNK)],
                                 rows[j], gsem[j])

            def outer(i, c2):
                for j in range(NBUF):
                    k = i * NBUF + j
                    j2 = (j + LEAD) % NBUF
                    pltpu.make_async_copy(
                        efeat_h.at[pl.ds(base + k * CHUNK, CHUNK)],
                        rows[j], gsem[j]).wait()
                    pltpu.async_copy(rows[j], acc_sh.at[didx.at[k]],
                                     ssem[j], add=True)
                    pltpu.async_copy(ones, cnt_sh.at[didx.at[k]], csem,
                                     add=True)

                    @pl.when(k + LEAD < BLK_CH)
                    def _refill():
                        @pl.when(k >= LEAD)
                        def _():
                            pltpu.make_async_copy(rows[j2],
                                                  acc_sh.at[didx.at[0]],
                                                  ssem[j2]).wait()
                        pltpu.async_copy(
                            efeat_h.at[pl.ds(base + (k + LEAD) * CHUNK,
                                             CHUNK)],
                            rows[j2], gsem[j2])
                return c2
            lax.fori_loop(0, OUTER, outer, 0)

            for j in range(NBUF):
                pltpu.make_async_copy(rows[j], acc_sh.at[didx.at[0]],
                                      ssem[j]).wait()

            def _cdrain(k, c2):
                pltpu.make_async_copy(ones, cnt_sh.at[didx.at[0]],
                                      csem).wait()
                return c2
            lax.fori_loop(0, BLK_CH, _cdrain, 0)
            return carry
        lax.fori_loop(0, NBLK, blk, 0)

        pltpu.sync_copy(tdst_h.at[sid], tdidx)
        pltpu.sync_copy(efeat_h.at[pl.ds(sid * EPT + MAIN, TAIL)], r0)
        pltpu.sync_copy(r0, acc_sh.at[tdidx.at[0]], add=True)
        pltpu.sync_copy(ones, cnt_sh.at[tdidx.at[0]], add=True)

    plsc.subcore_barrier()

    @pl.when(cid == 0)
    def _out0():
        pltpu.sync_copy(acc_sh.at[pl.ds(sid * ROWS_PT, ROWS_PT)],
                        agg_o.at[pl.ds(sid * ROWS_PT, ROWS_PT)])

    @pl.when(cid == 1)
    def _out1():
        pltpu.sync_copy(acc_sh.at[pl.ds(sid * ROWS_PT, ROWS_PT)],
                        esum_o.at[pl.ds(sid * ROWS_PT, ROWS_PT)])
        pltpu.sync_copy(cnt_sh.at[pl.ds(sid * CPT, CPT)],
                        cnt_o.at[pl.ds(sid * CPT, CPT)])


_sc_segsum = functools.partial(
    pl.kernel,
    out_type=(
        jax.ShapeDtypeStruct((NPAD, D), jnp.float32),
        jax.ShapeDtypeStruct((NPAD, D), jnp.float32),
        jax.ShapeDtypeStruct((CNT_PAD,), jnp.float32),
    ),
    mesh=plsc.VectorSubcoreMesh(core_axis_name="c", subcore_axis_name="s"),
    scratch_types=[
        pltpu.VMEM((BLK_CH, CHUNK), jnp.int32),
        pltpu.VMEM((BLK_CH, CHUNK), jnp.int32),
        pltpu.VMEM((1, CHUNK), jnp.int32),
        pltpu.VMEM((1, CHUNK), jnp.int32),
        pltpu.VMEM((CHUNK, D), jnp.float32),
        pltpu.VMEM((CHUNK, D), jnp.float32),
        pltpu.VMEM((CHUNK, D), jnp.float32),
        pltpu.VMEM((CHUNK, D), jnp.float32),
        pltpu.VMEM((CHUNK, D), jnp.float32),
        pltpu.VMEM((CHUNK, D), jnp.float32),
        pltpu.VMEM((CHUNK, D), jnp.float32),
        pltpu.VMEM((CHUNK, D), jnp.float32),
        pltpu.VMEM((CHUNK,), jnp.float32),
        pltpu.VMEM((CPT,), jnp.float32),
        pltpu.VMEM_SHARED((NPAD, D), jnp.float32),
        pltpu.VMEM_SHARED((CNT_PAD,), jnp.float32),
        pltpu.SemaphoreType.DMA,
        pltpu.SemaphoreType.DMA,
        pltpu.SemaphoreType.DMA,
        pltpu.SemaphoreType.DMA,
        pltpu.SemaphoreType.DMA,
        pltpu.SemaphoreType.DMA,
        pltpu.SemaphoreType.DMA,
        pltpu.SemaphoreType.DMA,
        pltpu.SemaphoreType.DMA,
        pltpu.SemaphoreType.DMA,
        pltpu.SemaphoreType.DMA,
        pltpu.SemaphoreType.DMA,
        pltpu.SemaphoreType.DMA,
        pltpu.SemaphoreType.DMA,
        pltpu.SemaphoreType.DMA,
        pltpu.SemaphoreType.DMA,
        pltpu.SemaphoreType.DMA,
    ],
)(_sc_body)


ROWS_BLK = 1000


def _tc_body(x_ref, a_ref, s_ref, c_ref, w_ref, b_ref, we_ref, be_ref, o_ref):
    c = c_ref[...]
    inv_d = 1.0 / (c + 1.0)
    inv_m = 1.0 / jnp.maximum(c, 1.0)
    x = a_ref[...] + x_ref[...] * inv_d
    y = s_ref[...] * inv_m
    out = jnp.dot(x, w_ref[...], preferred_element_type=jnp.float32)
    out += jnp.dot(y, we_ref[...], preferred_element_type=jnp.float32)
    out += b_ref[...] * (1.0 + inv_d)
    out += be_ref[...] * (c * inv_m)
    o_ref[...] = out


def _tc_combine(nfeat, agg_p, esum_p, cnt, W, b, We, be):
    grid = N_NODES // ROWS_BLK
    return pl.pallas_call(
        _tc_body,
        grid=(grid,),
        in_specs=[
            pl.BlockSpec((ROWS_BLK, D), lambda i: (i, 0)),
            pl.BlockSpec((ROWS_BLK, D), lambda i: (i, 0)),
            pl.BlockSpec((ROWS_BLK, D), lambda i: (i, 0)),
            pl.BlockSpec((ROWS_BLK, 1), lambda i: (i, 0)),
            pl.BlockSpec((D, D), lambda i: (0, 0)),
            pl.BlockSpec((1, D), lambda i: (0, 0)),
            pl.BlockSpec((D, D), lambda i: (0, 0)),
            pl.BlockSpec((1, D), lambda i: (0, 0)),
        ],
        out_specs=pl.BlockSpec((ROWS_BLK, D), lambda i: (i, 0)),
        out_shape=jax.ShapeDtypeStruct((N_NODES, D), jnp.float32),
    )(nfeat, agg_p, esum_p, cnt, W, b, We, be)


def kernel(nfeat, edge_index, efeat, W, b, We, be):
    src = edge_index[0].astype(jnp.int32).reshape(NS, EPT)
    dst = edge_index[1].astype(jnp.int32).reshape(NS, EPT)
    src_m = src[:, :MAIN].reshape(NS * NBLK, BLK_CH, CHUNK)
    dst_m = dst[:, :MAIN].reshape(NS * NBLK, BLK_CH, CHUNK)
    src_t = src[:, MAIN:].reshape(NS, 1, TAIL)
    dst_t = dst[:, MAIN:].reshape(NS, 1, TAIL)
    agg_p, esum_p, cnt_pad = _sc_segsum(src_m, dst_m, src_t, dst_t,
                                        nfeat, efeat)
    cnt = cnt_pad[:N_NODES].reshape(N_NODES, 1)
    return _tc_combine(nfeat, agg_p, esum_p, cnt,
                       W, b.reshape(1, D), We, be.reshape(1, D))
